# baseline (device time: 1406339 ns/iter reference)
import jax
import jax.numpy as jnp
from jax import lax
from jax.experimental import pallas as pl
from jax.experimental.pallas import tpu as pltpu

N_DEV = 8
M, K, N = 8192, 1024, 4096
CH = M // N_DEV
HALF = N // 2
SUB = CH // 2


def _gelu(v):
    c = 0.7978845608028654
    return 0.5 * v * (1.0 + jnp.tanh(c * (v + 0.044715 * v * v * v)))


def _body(x_ref, w_ref, acc_ref, o_ref, comm_ref,
          sv, vp, vx,
          rs_send, rs_recv, ag_send, ag_recv, cp_sem):
    me = lax.axis_index("i")
    right = lax.rem(me + 1, N_DEV)
    left = lax.rem(me + N_DEV - 1, N_DEV)

    barrier = pltpu.get_barrier_semaphore()
    for nbr in (left, right):
        pl.semaphore_signal(
            barrier, inc=1, device_id=(nbr,),
            device_id_type=pl.DeviceIdType.MESH,
        )

    def xchunk(idx):
        return x_ref.at[pl.ds(idx * CH, CH), :]

    def wcols(d):
        return w_ref[:, d * HALF:(d + 1) * HALF]

    def osub(idx, d, h):
        return o_ref.at[pl.ds(idx * CH + h * SUB, SUB),
                        pl.ds(d * HALF, HALF)]

    dirs = ((0, 1, right), (1, -1, left))
    rings = tuple((h,) + dd for h in (0, 1) for dd in dirs)

    def rs_rdma(d, h, tgt, s):
        return pltpu.make_async_remote_copy(
            src_ref=sv.at[d, pl.ds(h * SUB, SUB), :],
            dst_ref=comm_ref.at[d, s, pl.ds(h * SUB, SUB), :],
            send_sem=rs_send.at[d, s, h],
            recv_sem=rs_recv.at[d, s, h],
            device_id=(tgt,),
            device_id_type=pl.DeviceIdType.MESH,
        )

    c0 = pltpu.make_async_copy(xchunk(me), vx.at[0], cp_sem.at[4])
    c0.start()
    c0.wait()
    inflight = {}
    first = True
    for h, d, disp, tgt in rings:
        sv[d, h * SUB:(h + 1) * SUB, :] = jnp.dot(
            vx[0, h * SUB:(h + 1) * SUB, :], wcols(d),
            preferred_element_type=jnp.float32)
        if first:
            pl.semaphore_wait(barrier, 2)
            first = False
        rdma = rs_rdma(d, h, tgt, 0)
        rdma.start()
        inflight[(d, h)] = rdma

    def mm_partials(s):
        xc = []
        for d, disp, tgt in dirs:
            idx = lax.rem(me - disp * (s + 1) + 2 * N_DEV, N_DEV)
            c = pltpu.make_async_copy(xchunk(idx), vx.at[d], cp_sem.at[4 + d])
            c.start()
            xc.append(c)
        for d, disp, tgt in dirs:
            xc[d].wait()
            vp[d, :, :] = jnp.dot(
                vx[d], wcols(d), preferred_element_type=jnp.float32)

    mm_partials(0)
    for s in range(1, N_DEV - 1):
        nxt = {}
        for h, d, disp, tgt in rings:
            inflight[(d, h)].wait()
            c1 = pltpu.make_async_copy(
                comm_ref.at[d, s - 1, pl.ds(h * SUB, SUB), :],
                sv.at[d, pl.ds(h * SUB, SUB), :],
                cp_sem.at[2 * d + h])
            c1.start()
            c1.wait()
            rows = slice(h * SUB, (h + 1) * SUB)
            sv[d, rows, :] = sv[d, rows, :] + vp[d, rows, :]
            rdma = rs_rdma(d, h, tgt, s)
            rdma.start()
            nxt[(d, h)] = rdma
        inflight = nxt
        mm_partials(s)

    stores = []
    ag_inflight = {}
    for h, d, disp, tgt in rings:
        r = lax.rem(me + disp + N_DEV, N_DEV)
        inflight[(d, h)].wait()
        c1 = pltpu.make_async_copy(
            comm_ref.at[d, N_DEV - 2, pl.ds(h * SUB, SUB), :],
            sv.at[d, pl.ds(h * SUB, SUB), :],
            cp_sem.at[2 * d + h])
        c1.start()
        c1.wait()
        rows = slice(h * SUB, (h + 1) * SUB)
        sv[d, rows, :] = _gelu(sv[d, rows, :] + vp[d, rows, :])
        rdma = pltpu.make_async_remote_copy(
            src_ref=sv.at[d, pl.ds(h * SUB, SUB), :],
            dst_ref=osub(r, d, h),
            send_sem=ag_send.at[d, 0, h],
            recv_sem=ag_recv.at[d, 0, h],
            device_id=(tgt,),
            device_id_type=pl.DeviceIdType.MESH,
        )
        rdma.start()
        ag_inflight[(d, h)] = rdma
        st = pltpu.make_async_copy(
            sv.at[d, pl.ds(h * SUB, SUB), :], osub(r, d, h),
            cp_sem.at[8 + 2 * d + h])
        st.start()
        stores.append(st)

    for s in range(1, N_DEV - 1):
        nxt = {}
        for h, d, disp, tgt in rings:
            ag_inflight[(d, h)].wait()
            idx = lax.rem(me + disp - disp * s + 2 * N_DEV, N_DEV)
            rdma = pltpu.make_async_remote_copy(
                src_ref=osub(idx, d, h),
                dst_ref=osub(idx, d, h),
                send_sem=ag_send.at[d, s, h],
                recv_sem=ag_recv.at[d, s, h],
                device_id=(tgt,),
                device_id_type=pl.DeviceIdType.MESH,
            )
            rdma.start()
            nxt[(d, h)] = rdma
        ag_inflight = nxt
    for rdma in ag_inflight.values():
        rdma.wait()
    for st in stores:
        st.wait()


def kernel(x, w_mat):
    acc = jnp.zeros((M, N), jnp.float32)
    out, _ = pl.pallas_call(
        _body,
        in_specs=[
            pl.BlockSpec(memory_space=pltpu.HBM),
            pl.BlockSpec(memory_space=pltpu.VMEM),
            pl.BlockSpec(memory_space=pltpu.HBM),
        ],
        input_output_aliases={2: 0},
        out_specs=[
            pl.BlockSpec(memory_space=pltpu.HBM),
            pl.BlockSpec(memory_space=pltpu.HBM),
        ],
        out_shape=[
            jax.ShapeDtypeStruct((M, N), jnp.float32),
            jax.ShapeDtypeStruct((2, N_DEV - 1, CH, HALF), jnp.float32),
        ],
        scratch_shapes=[
            pltpu.VMEM((2, CH, HALF), jnp.float32),
            pltpu.VMEM((2, CH, HALF), jnp.float32),
            pltpu.VMEM((2, CH, K), jnp.float32),
            pltpu.SemaphoreType.DMA((2, N_DEV - 1, 2)),
            pltpu.SemaphoreType.DMA((2, N_DEV - 1, 2)),
            pltpu.SemaphoreType.DMA((2, N_DEV - 1, 2)),
            pltpu.SemaphoreType.DMA((2, N_DEV - 1, 2)),
            pltpu.SemaphoreType.DMA((12,)),
        ],
        compiler_params=pltpu.CompilerParams(
            collective_id=0, vmem_limit_bytes=100 * 1024 * 1024
        ),
    )(x, w_mat, acc)
    return out


# device time: 1362063 ns/iter; 1.0325x vs baseline; 1.0325x over previous
import jax
import jax.numpy as jnp
from jax import lax
from jax.experimental import pallas as pl
from jax.experimental.pallas import tpu as pltpu

N_DEV = 8
M, K, N = 8192, 1024, 4096
CH = M // N_DEV
HALF = N // 2
SUB = CH // 2


def _gelu(v):
    c = 0.7978845608028654
    return 0.5 * v * (1.0 + jnp.tanh(c * (v + 0.044715 * v * v * v)))


def _body(x_ref, w_ref, o_ref, comm_ref,
          sv, vp, vx,
          rs_send, rs_recv, ag_send, ag_recv, cp_sem):
    me = lax.axis_index("i")
    right = lax.rem(me + 1, N_DEV)
    left = lax.rem(me + N_DEV - 1, N_DEV)

    barrier = pltpu.get_barrier_semaphore()
    for nbr in (left, right):
        pl.semaphore_signal(
            barrier, inc=1, device_id=(nbr,),
            device_id_type=pl.DeviceIdType.MESH,
        )

    def xchunk(idx):
        return x_ref.at[pl.ds(idx * CH, CH), :]

    def wcols(d):
        return w_ref[:, d * HALF:(d + 1) * HALF]

    def osub(idx, d, h):
        return o_ref.at[pl.ds(idx * CH + h * SUB, SUB),
                        pl.ds(d * HALF, HALF)]

    dirs = ((0, 1, right), (1, -1, left))
    rings = tuple((h,) + dd for h in (0, 1) for dd in dirs)

    def rs_rdma(d, h, tgt, s):
        return pltpu.make_async_remote_copy(
            src_ref=sv.at[d, pl.ds(h * SUB, SUB), :],
            dst_ref=comm_ref.at[d, s, pl.ds(h * SUB, SUB), :],
            send_sem=rs_send.at[d, s, h],
            recv_sem=rs_recv.at[d, s, h],
            device_id=(tgt,),
            device_id_type=pl.DeviceIdType.MESH,
        )

    c0 = pltpu.make_async_copy(xchunk(me), vx.at[0], cp_sem.at[4])
    c0.start()
    c0.wait()
    inflight = {}
    first = True
    for h, d, disp, tgt in rings:
        sv[d, h * SUB:(h + 1) * SUB, :] = jnp.dot(
            vx[0, h * SUB:(h + 1) * SUB, :], wcols(d),
            preferred_element_type=jnp.float32)
        if first:
            pl.semaphore_wait(barrier, 2)
            first = False
        rdma = rs_rdma(d, h, tgt, 0)
        rdma.start()
        inflight[(d, h)] = rdma

    def mm_partials(s):
        xc = []
        for d, disp, tgt in dirs:
            idx = lax.rem(me - disp * (s + 1) + 2 * N_DEV, N_DEV)
            c = pltpu.make_async_copy(xchunk(idx), vx.at[d], cp_sem.at[4 + d])
            c.start()
            xc.append(c)
        for d, disp, tgt in dirs:
            xc[d].wait()
            vp[d, :, :] = jnp.dot(
                vx[d], wcols(d), preferred_element_type=jnp.float32)

    mm_partials(0)
    for s in range(1, N_DEV - 1):
        nxt = {}
        for h, d, disp, tgt in rings:
            inflight[(d, h)].wait()
            c1 = pltpu.make_async_copy(
                comm_ref.at[d, s - 1, pl.ds(h * SUB, SUB), :],
                sv.at[d, pl.ds(h * SUB, SUB), :],
                cp_sem.at[2 * d + h])
            c1.start()
            c1.wait()
            rows = slice(h * SUB, (h + 1) * SUB)
            sv[d, rows, :] = sv[d, rows, :] + vp[d, rows, :]
            rdma = rs_rdma(d, h, tgt, s)
            rdma.start()
            nxt[(d, h)] = rdma
        inflight = nxt
        mm_partials(s)

    stores = []
    ag_inflight = {}
    for h, d, disp, tgt in rings:
        r = lax.rem(me + disp + N_DEV, N_DEV)
        inflight[(d, h)].wait()
        c1 = pltpu.make_async_copy(
            comm_ref.at[d, N_DEV - 2, pl.ds(h * SUB, SUB), :],
            sv.at[d, pl.ds(h * SUB, SUB), :],
            cp_sem.at[2 * d + h])
        c1.start()
        c1.wait()
        rows = slice(h * SUB, (h + 1) * SUB)
        sv[d, rows, :] = _gelu(sv[d, rows, :] + vp[d, rows, :])
        rdma = pltpu.make_async_remote_copy(
            src_ref=sv.at[d, pl.ds(h * SUB, SUB), :],
            dst_ref=osub(r, d, h),
            send_sem=ag_send.at[d, 0, h],
            recv_sem=ag_recv.at[d, 0, h],
            device_id=(tgt,),
            device_id_type=pl.DeviceIdType.MESH,
        )
        rdma.start()
        ag_inflight[(d, h)] = rdma
        st = pltpu.make_async_copy(
            sv.at[d, pl.ds(h * SUB, SUB), :], osub(r, d, h),
            cp_sem.at[8 + 2 * d + h])
        st.start()
        stores.append(st)

    for s in range(1, N_DEV - 1):
        nxt = {}
        for h, d, disp, tgt in rings:
            ag_inflight[(d, h)].wait()
            idx = lax.rem(me + disp - disp * s + 2 * N_DEV, N_DEV)
            rdma = pltpu.make_async_remote_copy(
                src_ref=osub(idx, d, h),
                dst_ref=osub(idx, d, h),
                send_sem=ag_send.at[d, s, h],
                recv_sem=ag_recv.at[d, s, h],
                device_id=(tgt,),
                device_id_type=pl.DeviceIdType.MESH,
            )
            rdma.start()
            nxt[(d, h)] = rdma
        ag_inflight = nxt
    for rdma in ag_inflight.values():
        rdma.wait()
    for st in stores:
        st.wait()


def kernel(x, w_mat):
    out, _ = pl.pallas_call(
        _body,
        in_specs=[
            pl.BlockSpec(memory_space=pltpu.HBM),
            pl.BlockSpec(memory_space=pltpu.VMEM),
        ],
        out_specs=[
            pl.BlockSpec(memory_space=pltpu.HBM),
            pl.BlockSpec(memory_space=pltpu.HBM),
        ],
        out_shape=[
            jax.ShapeDtypeStruct((M, N), jnp.float32),
            jax.ShapeDtypeStruct((2, N_DEV - 1, CH, HALF), jnp.float32),
        ],
        scratch_shapes=[
            pltpu.VMEM((2, CH, HALF), jnp.float32),
            pltpu.VMEM((2, CH, HALF), jnp.float32),
            pltpu.VMEM((2, CH, K), jnp.float32),
            pltpu.SemaphoreType.DMA((2, N_DEV - 1, 2)),
            pltpu.SemaphoreType.DMA((2, N_DEV - 1, 2)),
            pltpu.SemaphoreType.DMA((2, N_DEV - 1, 2)),
            pltpu.SemaphoreType.DMA((2, N_DEV - 1, 2)),
            pltpu.SemaphoreType.DMA((12,)),
        ],
        compiler_params=pltpu.CompilerParams(
            collective_id=0, vmem_limit_bytes=100 * 1024 * 1024
        ),
    )(x, w_mat)
    return out
